# baseline (speedup 1.0000x reference)
import jax
import jax.numpy as jnp
from jax import lax
from jax.experimental import pallas as pl
from jax.experimental.pallas import tpu as pltpu

ROWS = 16384
DIM = 64
NCODES = 1024
TILE = 4096


def _vq_kernel(x_ref, e_ref, q_ref, ind_ref, dsum_ref):
    x = x_ref[...]
    e = e_ref[...]
    xe = jnp.dot(x, e, preferred_element_type=jnp.float32)
    e2 = jnp.sum(e * e, axis=0, keepdims=True)
    x2 = jnp.sum(x * x, axis=1, keepdims=True)
    dist = x2 - 2.0 * xe + e2
    m = jnp.min(dist, axis=1, keepdims=True)
    ind = jnp.argmin(dist, axis=1)
    iota = lax.broadcasted_iota(jnp.int32, (TILE, NCODES), 1)
    onehot = (iota == ind[:, None]).astype(jnp.bfloat16)
    q = lax.dot_general(
        onehot, e.astype(jnp.bfloat16), (((1,), (1,)), ((), ())),
        preferred_element_type=jnp.float32,
    )
    q_ref[...] = q
    ind_ref[...] = ind[:, None]
    dsum_ref[...] = jnp.sum(m, keepdims=True)[None]


def kernel(input_, embed):
    grid = (ROWS // TILE,)
    q, ind, dsum = pl.pallas_call(
        _vq_kernel,
        grid=grid,
        in_specs=[
            pl.BlockSpec((TILE, DIM), lambda i: (i, 0)),
            pl.BlockSpec((DIM, NCODES), lambda i: (0, 0)),
        ],
        out_specs=[
            pl.BlockSpec((TILE, DIM), lambda i: (i, 0)),
            pl.BlockSpec((TILE, 1), lambda i: (i, 0)),
            pl.BlockSpec((1, 1, 1), lambda i: (i, 0, 0)),
        ],
        compiler_params=pltpu.CompilerParams(vmem_limit_bytes=120 * 1024 * 1024),
        out_shape=[
            jax.ShapeDtypeStruct((ROWS, DIM), jnp.float32),
            jax.ShapeDtypeStruct((ROWS, 1), jnp.int32),
            jax.ShapeDtypeStruct((ROWS // TILE, 1, 1), jnp.float32),
        ],
    )(input_, embed)
    diff = jnp.sum(dsum) / (ROWS * DIM)
    return q, diff, ind.reshape(-1)


# speedup vs baseline: 1.3335x; 1.3335x over previous
import jax
import jax.numpy as jnp
from jax import lax
from jax.experimental import pallas as pl
from jax.experimental.pallas import tpu as pltpu

ROWS = 16384
DIM = 64
NCODES = 1024
TILE = 4096


def _vq_kernel(x_ref, e_ref, q_ref, ind_ref, dsum_ref):
    x = x_ref[...]
    e = e_ref[...]
    xe = jnp.dot(x, e, preferred_element_type=jnp.float32)
    e2 = jnp.sum(e * e, axis=0, keepdims=True)
    x2 = jnp.sum(x * x, axis=1, keepdims=True)
    dist = x2 - 2.0 * xe + e2
    m = jnp.min(dist, axis=1, keepdims=True)
    iota = lax.broadcasted_iota(jnp.int32, (TILE, NCODES), 1)
    ind = jnp.min(jnp.where(dist == m, iota, NCODES), axis=1)
    onehot = (iota == ind[:, None]).astype(jnp.bfloat16)
    q = lax.dot_general(
        onehot, e.astype(jnp.bfloat16), (((1,), (1,)), ((), ())),
        preferred_element_type=jnp.float32,
    )
    q_ref[...] = q
    ind_ref[...] = ind[:, None]
    dsum_ref[...] = jnp.sum(m, keepdims=True)[None]


def kernel(input_, embed):
    grid = (ROWS // TILE,)
    q, ind, dsum = pl.pallas_call(
        _vq_kernel,
        grid=grid,
        in_specs=[
            pl.BlockSpec((TILE, DIM), lambda i: (i, 0)),
            pl.BlockSpec((DIM, NCODES), lambda i: (0, 0)),
        ],
        out_specs=[
            pl.BlockSpec((TILE, DIM), lambda i: (i, 0)),
            pl.BlockSpec((TILE, 1), lambda i: (i, 0)),
            pl.BlockSpec((1, 1, 1), lambda i: (i, 0, 0)),
        ],
        compiler_params=pltpu.CompilerParams(vmem_limit_bytes=120 * 1024 * 1024),
        out_shape=[
            jax.ShapeDtypeStruct((ROWS, DIM), jnp.float32),
            jax.ShapeDtypeStruct((ROWS, 1), jnp.int32),
            jax.ShapeDtypeStruct((ROWS // TILE, 1, 1), jnp.float32),
        ],
    )(input_, embed)
    diff = jnp.sum(dsum) / (ROWS * DIM)
    return q, diff, ind.reshape(-1)
